# Initial kernel scaffold; baseline (speedup 1.0000x reference)
import functools

import jax
import jax.numpy as jnp
from jax import lax
from jax.experimental import pallas as pl
from jax.experimental.pallas import tpu as pltpu
from jax.experimental.pallas import tpu_sc as plsc

N = 10000
E = 320000
D_IN = 128
D_HID = 64
D_OUT = 128

NC = 2
NS = 16
NW = NC * NS

K = 128
NCHUNK = 2528
EP = NCHUNK * K
CPT = NCHUNK // NW

NP = 10240
RPT = NP // NS

R = 256
GRID = NP // R


def _deg_kernel(dst_hbm, ones_hbm, z_hbm, out_hbm, didx, ones_v, acc, sem):
  c = lax.axis_index("c")
  s = lax.axis_index("s")
  wid = c * NS + s
  pltpu.sync_copy(z_hbm.at[pl.ds(s * RPT, RPT)], acc.at[pl.ds(s * RPT, RPT)])
  pltpu.sync_copy(ones_hbm, ones_v)
  pltpu.sync_copy(dst_hbm.at[pl.ds(wid * CPT, CPT)], didx)
  plsc.subcore_barrier()

  def body(j, carry):
    pltpu.sync_copy(ones_v, acc.at[didx.at[j]], add=True)
    return carry

  lax.fori_loop(0, CPT, body, 0)
  plsc.subcore_barrier()
  pltpu.sync_copy(acc.at[pl.ds(s * RPT, RPT)],
                  out_hbm.at[c].at[pl.ds(s * RPT, RPT)])


def _make_deg():
  mesh = plsc.VectorSubcoreMesh(core_axis_name="c", subcore_axis_name="s")
  return pl.kernel(
      _deg_kernel,
      out_type=jax.ShapeDtypeStruct((NC, NP, 16), jnp.float32),
      mesh=mesh,
      scratch_types=[
          pltpu.VMEM((CPT, K), jnp.int32),
          pltpu.VMEM((K, 16), jnp.float32),
          pltpu.VMEM_SHARED((NP, 16), jnp.float32),
          pltpu.SemaphoreType.DMA,
      ],
  )


def _agg_kernel(y_hbm, src_hbm, dst_hbm, z_hbm, out_hbm,
                sidx, didx, msg, acc, sem):
  c = lax.axis_index("c")
  s = lax.axis_index("s")
  wid = c * NS + s
  pltpu.sync_copy(z_hbm.at[pl.ds(s * RPT, RPT)], acc.at[pl.ds(s * RPT, RPT)])
  pltpu.sync_copy(src_hbm.at[pl.ds(wid * CPT, CPT)], sidx)
  pltpu.sync_copy(dst_hbm.at[pl.ds(wid * CPT, CPT)], didx)
  plsc.subcore_barrier()

  def body(j, carry):
    pltpu.async_copy(y_hbm.at[sidx.at[j]], msg, sem).wait()
    pltpu.sync_copy(msg, acc.at[didx.at[j]], add=True)
    return carry

  lax.fori_loop(0, CPT, body, 0)
  plsc.subcore_barrier()
  pltpu.sync_copy(acc.at[pl.ds(s * RPT, RPT)],
                  out_hbm.at[c].at[pl.ds(s * RPT, RPT)])


def _make_agg(d):
  mesh = plsc.VectorSubcoreMesh(core_axis_name="c", subcore_axis_name="s")
  return pl.kernel(
      _agg_kernel,
      out_type=jax.ShapeDtypeStruct((NC, NP, d), jnp.float32),
      mesh=mesh,
      scratch_types=[
          pltpu.VMEM((CPT, K), jnp.int32),
          pltpu.VMEM((CPT, K), jnp.int32),
          pltpu.VMEM((K, d), jnp.float32),
          pltpu.VMEM_SHARED((NP, d), jnp.float32),
          pltpu.SemaphoreType.DMA,
      ],
  )


def _y1_kernel(x_ref, w_ref, degp_ref, y1_ref, dis_ref):
  deg = degp_ref[0, :, 0] + degp_ref[1, :, 0] + 1.0
  dis = lax.rsqrt(deg)
  y = jnp.dot(x_ref[...], w_ref[...], preferred_element_type=jnp.float32)
  y1_ref[...] = y * dis[:, None]
  dis_ref[...] = dis


def _mid_kernel(aggp_ref, y1_ref, dis_ref, b1_ref, w2_ref, y2_ref):
  dis = dis_ref[...]
  a = aggp_ref[0] + aggp_ref[1] + y1_ref[...]
  h = jnp.maximum(a * dis[:, None] + b1_ref[...][None, :], 0.0)
  y2_ref[...] = jnp.dot(h, w2_ref[...],
                        preferred_element_type=jnp.float32) * dis[:, None]


def _out_kernel(aggp_ref, y2_ref, dis_ref, b2_ref, o_ref):
  a = aggp_ref[0] + aggp_ref[1] + y2_ref[...]
  o_ref[...] = a * dis_ref[...][:, None] + b2_ref[...][None, :]


def _tc_y1(x, w1, degp):
  return pl.pallas_call(
      _y1_kernel,
      grid=(GRID,),
      in_specs=[
          pl.BlockSpec((R, D_IN), lambda i: (i, 0)),
          pl.BlockSpec((D_IN, D_HID), lambda i: (0, 0)),
          pl.BlockSpec((NC, R, 16), lambda i: (0, i, 0)),
      ],
      out_specs=[
          pl.BlockSpec((R, D_HID), lambda i: (i, 0)),
          pl.BlockSpec((R,), lambda i: (i,)),
      ],
      out_shape=[
          jax.ShapeDtypeStruct((NP, D_HID), jnp.float32),
          jax.ShapeDtypeStruct((NP,), jnp.float32),
      ],
  )(x, w1, degp)


def _tc_mid(aggp, y1, dis, b1, w2):
  return pl.pallas_call(
      _mid_kernel,
      grid=(GRID,),
      in_specs=[
          pl.BlockSpec((NC, R, D_HID), lambda i: (0, i, 0)),
          pl.BlockSpec((R, D_HID), lambda i: (i, 0)),
          pl.BlockSpec((R,), lambda i: (i,)),
          pl.BlockSpec((D_HID,), lambda i: (0,)),
          pl.BlockSpec((D_HID, D_OUT), lambda i: (0, 0)),
      ],
      out_specs=pl.BlockSpec((R, D_OUT), lambda i: (i, 0)),
      out_shape=jax.ShapeDtypeStruct((NP, D_OUT), jnp.float32),
  )(aggp, y1, dis, b1, w2)


def _tc_out(aggp, y2, dis, b2):
  return pl.pallas_call(
      _out_kernel,
      grid=(GRID,),
      in_specs=[
          pl.BlockSpec((NC, R, D_OUT), lambda i: (0, i, 0)),
          pl.BlockSpec((R, D_OUT), lambda i: (i, 0)),
          pl.BlockSpec((R,), lambda i: (i,)),
          pl.BlockSpec((D_OUT,), lambda i: (0,)),
      ],
      out_specs=pl.BlockSpec((R, D_OUT), lambda i: (i, 0)),
      out_shape=jax.ShapeDtypeStruct((NP, D_OUT), jnp.float32),
  )(aggp, y2, dis, b2)


@jax.jit
def _run(x, edge_index, w1, b1, w2, b2):
  pad = (jnp.arange(EP - E, dtype=jnp.int32) % (NP - N)) + N
  src = jnp.concatenate([edge_index[0].astype(jnp.int32), pad]).reshape(
      NCHUNK, K)
  dst = jnp.concatenate([edge_index[1].astype(jnp.int32), pad]).reshape(
      NCHUNK, K)
  xp = jnp.zeros((NP, D_IN), jnp.float32).at[:N].set(x)

  ones16 = jnp.ones((K, 16), jnp.float32)
  z16 = jnp.zeros((NP, 16), jnp.float32)
  zh = jnp.zeros((NP, D_HID), jnp.float32)
  zo = jnp.zeros((NP, D_OUT), jnp.float32)

  degp = _make_deg()(dst, ones16, z16)
  y1, dis = _tc_y1(xp, w1, degp)
  agg1 = _make_agg(D_HID)(y1, src, dst, zh)
  y2 = _tc_mid(agg1, y1, dis, b1, w2)
  agg2 = _make_agg(D_OUT)(y2, src, dst, zo)
  out = _tc_out(agg2, y2, dis, b2)
  return out[:N]


def kernel(x, edge_index, W1, b1, W2, b2):
  return _run(x, edge_index, W1, b1, W2, b2)


# speedup vs baseline: 22.2336x; 22.2336x over previous
import functools

import jax
import jax.numpy as jnp
from jax import lax
from jax.experimental import pallas as pl
from jax.experimental.pallas import tpu as pltpu
from jax.experimental.pallas import tpu_sc as plsc

N = 10000
E = 320000
D_IN = 128
D_HID = 64
D_OUT = 128

NC = 2
NS = 16
NW = NC * NS

K = 128
NCHUNK = 2560
EP = NCHUNK * K
CPT = NCHUNK // NW

NP = 10240
RPT = NP // NS

R = 256
GRID = NP // R


def _deg_kernel(dst_hbm, ones_hbm, z_hbm, out_hbm, didx, ones_v, acc, sem):
  c = lax.axis_index("c")
  s = lax.axis_index("s")
  wid = c * NS + s
  pltpu.sync_copy(z_hbm.at[pl.ds(s * RPT, RPT)], acc.at[pl.ds(s * RPT, RPT)])
  pltpu.sync_copy(ones_hbm, ones_v)
  pltpu.sync_copy(dst_hbm.at[pl.ds(wid * CPT, CPT)], didx)
  plsc.subcore_barrier()

  def body(j, carry):
    pltpu.sync_copy(ones_v, acc.at[didx.at[j]], add=True)
    return carry

  lax.fori_loop(0, CPT, body, 0)
  plsc.subcore_barrier()
  pltpu.sync_copy(acc.at[pl.ds(s * RPT, RPT)],
                  out_hbm.at[c].at[pl.ds(s * RPT, RPT)])


def _make_deg():
  mesh = plsc.VectorSubcoreMesh(core_axis_name="c", subcore_axis_name="s",
                                num_cores=NC, num_subcores=NS)
  return pl.kernel(
      _deg_kernel,
      out_type=jax.ShapeDtypeStruct((NC, NP, 16), jnp.float32),
      mesh=mesh,
      scratch_types=[
          pltpu.VMEM((CPT, K), jnp.int32),
          pltpu.VMEM((K, 16), jnp.float32),
          pltpu.VMEM_SHARED((NP, 16), jnp.float32),
          pltpu.SemaphoreType.DMA,
      ],
      compiler_params=pltpu.CompilerParams(use_tc_tiling_on_sc=False),
  )


def _agg_kernel(y_hbm, src_hbm, dst_hbm, z_hbm, out_hbm,
                sidx, didx, msg, acc, sem):
  c = lax.axis_index("c")
  s = lax.axis_index("s")
  wid = c * NS + s
  pltpu.sync_copy(z_hbm.at[pl.ds(s * RPT, RPT)], acc.at[pl.ds(s * RPT, RPT)])
  pltpu.sync_copy(src_hbm.at[pl.ds(wid * CPT, CPT)], sidx)
  pltpu.sync_copy(dst_hbm.at[pl.ds(wid * CPT, CPT)], didx)
  plsc.subcore_barrier()

  def body(j, carry):
    pltpu.async_copy(y_hbm.at[sidx.at[j]], msg, sem).wait()
    pltpu.sync_copy(msg, acc.at[didx.at[j]], add=True)
    return carry

  lax.fori_loop(0, CPT, body, 0)
  plsc.subcore_barrier()
  pltpu.sync_copy(acc.at[pl.ds(s * RPT, RPT)],
                  out_hbm.at[c].at[pl.ds(s * RPT, RPT)])


def _make_agg(d):
  mesh = plsc.VectorSubcoreMesh(core_axis_name="c", subcore_axis_name="s",
                                num_cores=NC, num_subcores=NS)
  return pl.kernel(
      _agg_kernel,
      out_type=jax.ShapeDtypeStruct((NC, NP, d), jnp.float32),
      mesh=mesh,
      scratch_types=[
          pltpu.VMEM((CPT, K), jnp.int32),
          pltpu.VMEM((CPT, K), jnp.int32),
          pltpu.VMEM((K, d), jnp.float32),
          pltpu.VMEM_SHARED((NP, d), jnp.float32),
          pltpu.SemaphoreType.DMA,
      ],
      compiler_params=pltpu.CompilerParams(use_tc_tiling_on_sc=False),
  )


def _y1_kernel(x_ref, w_ref, degp_ref, y1_ref, dis_ref):
  deg = degp_ref[0, :, 0] + degp_ref[1, :, 0] + 1.0
  dis = lax.rsqrt(deg)
  y = jnp.dot(x_ref[...], w_ref[...], preferred_element_type=jnp.float32)
  y1_ref[...] = y * dis[:, None]
  dis_ref[...] = dis


def _mid_kernel(aggp_ref, y1_ref, dis_ref, b1_ref, w2_ref, y2_ref):
  dis = dis_ref[...]
  a = aggp_ref[0] + aggp_ref[1] + y1_ref[...]
  h = jnp.maximum(a * dis[:, None] + b1_ref[...][None, :], 0.0)
  y2_ref[...] = jnp.dot(h, w2_ref[...],
                        preferred_element_type=jnp.float32) * dis[:, None]


def _out_kernel(aggp_ref, y2_ref, dis_ref, b2_ref, o_ref):
  a = aggp_ref[0] + aggp_ref[1] + y2_ref[...]
  o_ref[...] = a * dis_ref[...][:, None] + b2_ref[...][None, :]


def _tc_y1(x, w1, degp):
  return pl.pallas_call(
      _y1_kernel,
      grid=(GRID,),
      in_specs=[
          pl.BlockSpec((R, D_IN), lambda i: (i, 0)),
          pl.BlockSpec((D_IN, D_HID), lambda i: (0, 0)),
          pl.BlockSpec((NC, R, 16), lambda i: (0, i, 0)),
      ],
      out_specs=[
          pl.BlockSpec((R, D_HID), lambda i: (i, 0)),
          pl.BlockSpec((R,), lambda i: (i,)),
      ],
      out_shape=[
          jax.ShapeDtypeStruct((NP, D_HID), jnp.float32),
          jax.ShapeDtypeStruct((NP,), jnp.float32),
      ],
  )(x, w1, degp)


def _tc_mid(aggp, y1, dis, b1, w2):
  return pl.pallas_call(
      _mid_kernel,
      grid=(GRID,),
      in_specs=[
          pl.BlockSpec((NC, R, D_HID), lambda i: (0, i, 0)),
          pl.BlockSpec((R, D_HID), lambda i: (i, 0)),
          pl.BlockSpec((R,), lambda i: (i,)),
          pl.BlockSpec((D_HID,), lambda i: (0,)),
          pl.BlockSpec((D_HID, D_OUT), lambda i: (0, 0)),
      ],
      out_specs=pl.BlockSpec((R, D_OUT), lambda i: (i, 0)),
      out_shape=jax.ShapeDtypeStruct((NP, D_OUT), jnp.float32),
  )(aggp, y1, dis, b1, w2)


def _tc_out(aggp, y2, dis, b2):
  return pl.pallas_call(
      _out_kernel,
      grid=(GRID,),
      in_specs=[
          pl.BlockSpec((NC, R, D_OUT), lambda i: (0, i, 0)),
          pl.BlockSpec((R, D_OUT), lambda i: (i, 0)),
          pl.BlockSpec((R,), lambda i: (i,)),
          pl.BlockSpec((D_OUT,), lambda i: (0,)),
      ],
      out_specs=pl.BlockSpec((R, D_OUT), lambda i: (i, 0)),
      out_shape=jax.ShapeDtypeStruct((NP, D_OUT), jnp.float32),
  )(aggp, y2, dis, b2)


@jax.jit
def _run(x, edge_index, w1, b1, w2, b2):
  pad = (jnp.arange(EP - E, dtype=jnp.int32) % (NP - N)) + N
  src = jnp.concatenate([edge_index[0].astype(jnp.int32), pad]).reshape(
      NCHUNK, K)
  dst = jnp.concatenate([edge_index[1].astype(jnp.int32), pad]).reshape(
      NCHUNK, K)
  xp = jnp.zeros((NP, D_IN), jnp.float32).at[:N].set(x)

  ones16 = jnp.ones((K, 16), jnp.float32)
  z16 = jnp.zeros((NP, 16), jnp.float32)
  zh = jnp.zeros((NP, D_HID), jnp.float32)
  zo = jnp.zeros((NP, D_OUT), jnp.float32)

  degp = _make_deg()(dst, ones16, z16)
  y1, dis = _tc_y1(xp, w1, degp)
  agg1 = _make_agg(D_HID)(y1, src, dst, zh)
  y2 = _tc_mid(agg1, y1, dis, b1, w2)
  agg2 = _make_agg(D_OUT)(y2, src, dst, zo)
  out = _tc_out(agg2, y2, dis, b2)
  return out[:N]


def kernel(x, edge_index, W1, b1, W2, b2):
  return _run(x, edge_index, W1, b1, W2, b2)


# speedup vs baseline: 25.4397x; 1.1442x over previous
import functools

import jax
import jax.numpy as jnp
from jax import lax
from jax.experimental import pallas as pl
from jax.experimental.pallas import tpu as pltpu
from jax.experimental.pallas import tpu_sc as plsc

N = 10000
E = 320000
D_IN = 128
D_HID = 64
D_OUT = 128

NC = 2
NS = 16
NW = NC * NS

K = 128
NCHUNK = 2560
EP = NCHUNK * K
CPT = NCHUNK // NW

NP = 10240
RPT = NP // NS

R = 256
GRID = NP // R


def _deg_kernel(dst_hbm, ones_hbm, z_hbm, out_hbm, didx, ones_v, acc, sem):
  c = lax.axis_index("c")
  s = lax.axis_index("s")
  wid = c * NS + s
  pltpu.sync_copy(z_hbm.at[pl.ds(s * RPT, RPT)], acc.at[pl.ds(s * RPT, RPT)])
  pltpu.sync_copy(ones_hbm, ones_v)
  pltpu.sync_copy(dst_hbm.at[pl.ds(wid * CPT, CPT)], didx)
  plsc.subcore_barrier()

  def body(j, carry):
    pltpu.sync_copy(ones_v, acc.at[didx.at[j]], add=True)
    return carry

  lax.fori_loop(0, CPT, body, 0)
  plsc.subcore_barrier()
  pltpu.sync_copy(acc.at[pl.ds(s * RPT, RPT)],
                  out_hbm.at[c].at[pl.ds(s * RPT, RPT)])


def _make_deg():
  mesh = plsc.VectorSubcoreMesh(core_axis_name="c", subcore_axis_name="s",
                                num_cores=NC, num_subcores=NS)
  return pl.kernel(
      _deg_kernel,
      out_type=jax.ShapeDtypeStruct((NC, NP, 16), jnp.float32),
      mesh=mesh,
      scratch_types=[
          pltpu.VMEM((CPT, K), jnp.int32),
          pltpu.VMEM((K, 16), jnp.float32),
          pltpu.VMEM_SHARED((NP, 16), jnp.float32),
          pltpu.SemaphoreType.DMA,
      ],
      compiler_params=pltpu.CompilerParams(use_tc_tiling_on_sc=False),
  )


def _agg_body(stage_y, y_hbm, src_hbm, dst_hbm, z_hbm, out_hbm,
              sb0, db0, sb1, db1, msg0, msg1, acc, y_sp,
              gs0, gs1, is0, id0, is1, id1):
  c = lax.axis_index("c")
  s = lax.axis_index("s")
  wid = c * NS + s
  rows = pl.ds(s * RPT, RPT)
  base = wid * CPT

  @pl.when(c == 0)
  def _():
    pltpu.sync_copy(y_hbm.at[rows], acc.at[rows])

  @pl.when(c != 0)
  def _():
    pltpu.sync_copy(z_hbm.at[rows], acc.at[rows])

  if stage_y:
    pltpu.sync_copy(y_hbm.at[rows], y_sp.at[rows])
    ysrc = y_sp
  else:
    ysrc = y_hbm
  pltpu.sync_copy(src_hbm.at[base], sb0)
  pltpu.sync_copy(dst_hbm.at[base], db0)
  plsc.subcore_barrier()
  pltpu.async_copy(ysrc.at[sb0], msg0, gs0)
  pltpu.async_copy(src_hbm.at[base + 1], sb1, is1)
  pltpu.async_copy(dst_hbm.at[base + 1], db1, id1)

  def body(t, carry):
    j0 = 2 * t
    j1 = j0 + 1
    pltpu.make_async_copy(ysrc.at[sb0], msg0, gs0).wait()
    pltpu.make_async_copy(src_hbm.at[base], sb1, is1).wait()
    pltpu.make_async_copy(dst_hbm.at[base], db1, id1).wait()
    pltpu.async_copy(ysrc.at[sb1], msg1, gs1)
    pltpu.sync_copy(msg0, acc.at[db0], add=True)

    @pl.when(j0 + 2 < CPT)
    def _():
      pltpu.async_copy(src_hbm.at[base + j0 + 2], sb0, is0)
      pltpu.async_copy(dst_hbm.at[base + j0 + 2], db0, id0)

    pltpu.make_async_copy(ysrc.at[sb1], msg1, gs1).wait()

    @pl.when(j0 + 2 < CPT)
    def _():
      pltpu.make_async_copy(src_hbm.at[base], sb0, is0).wait()
      pltpu.make_async_copy(dst_hbm.at[base], db0, id0).wait()
      pltpu.async_copy(ysrc.at[sb0], msg0, gs0)

    pltpu.sync_copy(msg1, acc.at[db1], add=True)

    @pl.when(j1 + 2 < CPT)
    def _():
      pltpu.async_copy(src_hbm.at[base + j1 + 2], sb1, is1)
      pltpu.async_copy(dst_hbm.at[base + j1 + 2], db1, id1)

    return carry

  lax.fori_loop(0, CPT // 2, body, 0)
  plsc.subcore_barrier()
  pltpu.sync_copy(acc.at[rows], out_hbm.at[c].at[rows])


def _make_agg(d, stage_y):
  mesh = plsc.VectorSubcoreMesh(core_axis_name="c", subcore_axis_name="s",
                                num_cores=NC, num_subcores=NS)
  return pl.kernel(
      functools.partial(_agg_body, stage_y),
      out_type=jax.ShapeDtypeStruct((NC, NP, d), jnp.float32),
      mesh=mesh,
      scratch_types=[
          pltpu.VMEM((K,), jnp.int32),
          pltpu.VMEM((K,), jnp.int32),
          pltpu.VMEM((K,), jnp.int32),
          pltpu.VMEM((K,), jnp.int32),
          pltpu.VMEM((K, d), jnp.float32),
          pltpu.VMEM((K, d), jnp.float32),
          pltpu.VMEM_SHARED((NP, d), jnp.float32),
          pltpu.VMEM_SHARED((NP, d) if stage_y else (8, d), jnp.float32),
          pltpu.SemaphoreType.DMA,
          pltpu.SemaphoreType.DMA,
          pltpu.SemaphoreType.DMA,
          pltpu.SemaphoreType.DMA,
          pltpu.SemaphoreType.DMA,
          pltpu.SemaphoreType.DMA,
      ],
      compiler_params=pltpu.CompilerParams(use_tc_tiling_on_sc=False),
  )


def _y1_kernel(x_ref, w_ref, degp_ref, y1_ref, dis_ref):
  deg = degp_ref[0, :, 0] + degp_ref[1, :, 0] + 1.0
  dis = lax.rsqrt(deg)
  y = jnp.dot(x_ref[...], w_ref[...], preferred_element_type=jnp.float32)
  y1_ref[...] = y * dis[:, None]
  dis_ref[...] = dis


def _mid_kernel(aggp_ref, dis_ref, b1_ref, w2_ref, y2_ref):
  dis = dis_ref[...]
  a = aggp_ref[0] + aggp_ref[1]
  h = jnp.maximum(a * dis[:, None] + b1_ref[...][None, :], 0.0)
  y2_ref[...] = jnp.dot(h, w2_ref[...],
                        preferred_element_type=jnp.float32) * dis[:, None]


def _out_kernel(aggp_ref, dis_ref, b2_ref, o_ref):
  a = aggp_ref[0] + aggp_ref[1]
  o_ref[...] = a * dis_ref[...][:, None] + b2_ref[...][None, :]


def _tc_y1(x, w1, degp):
  return pl.pallas_call(
      _y1_kernel,
      grid=(GRID,),
      in_specs=[
          pl.BlockSpec((R, D_IN), lambda i: (i, 0)),
          pl.BlockSpec((D_IN, D_HID), lambda i: (0, 0)),
          pl.BlockSpec((NC, R, 16), lambda i: (0, i, 0)),
      ],
      out_specs=[
          pl.BlockSpec((R, D_HID), lambda i: (i, 0)),
          pl.BlockSpec((R,), lambda i: (i,)),
      ],
      out_shape=[
          jax.ShapeDtypeStruct((NP, D_HID), jnp.float32),
          jax.ShapeDtypeStruct((NP,), jnp.float32),
      ],
  )(x, w1, degp)


def _tc_mid(aggp, dis, b1, w2):
  return pl.pallas_call(
      _mid_kernel,
      grid=(GRID,),
      in_specs=[
          pl.BlockSpec((NC, R, D_HID), lambda i: (0, i, 0)),
          pl.BlockSpec((R,), lambda i: (i,)),
          pl.BlockSpec((D_HID,), lambda i: (0,)),
          pl.BlockSpec((D_HID, D_OUT), lambda i: (0, 0)),
      ],
      out_specs=pl.BlockSpec((R, D_OUT), lambda i: (i, 0)),
      out_shape=jax.ShapeDtypeStruct((NP, D_OUT), jnp.float32),
  )(aggp, dis, b1, w2)


def _tc_out(aggp, dis, b2):
  return pl.pallas_call(
      _out_kernel,
      grid=(GRID,),
      in_specs=[
          pl.BlockSpec((NC, R, D_OUT), lambda i: (0, i, 0)),
          pl.BlockSpec((R,), lambda i: (i,)),
          pl.BlockSpec((D_OUT,), lambda i: (0,)),
      ],
      out_specs=pl.BlockSpec((R, D_OUT), lambda i: (i, 0)),
      out_shape=jax.ShapeDtypeStruct((NP, D_OUT), jnp.float32),
  )(aggp, dis, b2)


@jax.jit
def _run(x, edge_index, w1, b1, w2, b2):
  pad = (jnp.arange(EP - E, dtype=jnp.int32) % (NP - N)) + N
  src = jnp.concatenate([edge_index[0].astype(jnp.int32), pad]).reshape(
      NCHUNK, K)
  dst = jnp.concatenate([edge_index[1].astype(jnp.int32), pad]).reshape(
      NCHUNK, K)
  xp = jnp.zeros((NP, D_IN), jnp.float32).at[:N].set(x)

  ones16 = jnp.ones((K, 16), jnp.float32)
  z16 = jnp.zeros((NP, 16), jnp.float32)
  zh = jnp.zeros((NP, D_HID), jnp.float32)
  zo = jnp.zeros((NP, D_OUT), jnp.float32)

  degp = _make_deg()(dst, ones16, z16)
  y1, dis = _tc_y1(xp, w1, degp)
  agg1 = _make_agg(D_HID, True)(y1, src, dst, zh)
  y2 = _tc_mid(agg1, dis, b1, w2)
  agg2 = _make_agg(D_OUT, False)(y2, src, dst, zo)
  out = _tc_out(agg2, dis, b2)
  return out[:N]


def kernel(x, edge_index, W1, b1, W2, b2):
  return _run(x, edge_index, W1, b1, W2, b2)


# speedup vs baseline: 27.9657x; 1.0993x over previous
import functools

import jax
import jax.numpy as jnp
from jax import lax
from jax.experimental import pallas as pl
from jax.experimental.pallas import tpu as pltpu
from jax.experimental.pallas import tpu_sc as plsc

N = 10000
E = 320000
D_IN = 128
D_HID = 64
D_OUT = 128

NC = 2
NS = 16
NW = NC * NS

EP = 327680
NP = 10240
RPT = NP // NS

R = 256
GRID = NP // R


def _deg_kernel(dst_hbm, ones_hbm, z_hbm, out_hbm, didx, ones_v, acc, sem):
  kc = 512
  cpt = EP // kc // NW
  c = lax.axis_index("c")
  s = lax.axis_index("s")
  wid = c * NS + s
  rows = pl.ds(s * RPT, RPT)
  pltpu.sync_copy(z_hbm.at[rows], acc.at[rows])
  pltpu.sync_copy(ones_hbm, ones_v)
  plsc.subcore_barrier()

  def body(j, carry):
    pltpu.sync_copy(dst_hbm.at[pl.ds((wid * cpt + j) * kc, kc)], didx)
    pltpu.sync_copy(ones_v, acc.at[didx], add=True)
    return carry

  lax.fori_loop(0, cpt, body, 0)
  plsc.subcore_barrier()
  pltpu.sync_copy(acc.at[rows], out_hbm.at[c].at[rows])


def _make_deg():
  mesh = plsc.VectorSubcoreMesh(core_axis_name="c", subcore_axis_name="s",
                                num_cores=NC, num_subcores=NS)
  return pl.kernel(
      _deg_kernel,
      out_type=jax.ShapeDtypeStruct((NC, NP, 16), jnp.float32),
      mesh=mesh,
      scratch_types=[
          pltpu.VMEM((512,), jnp.int32),
          pltpu.VMEM((512, 16), jnp.float32),
          pltpu.VMEM_SHARED((NP, 16), jnp.float32),
          pltpu.SemaphoreType.DMA,
      ],
      compiler_params=pltpu.CompilerParams(use_tc_tiling_on_sc=False),
  )


def _agg_body(d, kc, y_hbm, src_hbm, dst_hbm, z_hbm, out_hbm,
              sb0, db0, sb1, db1, msg0, msg1, acc,
              gs0, gs1, is0, id0, is1, id1, ss0, ss1):
  cpt = EP // kc // NW
  c = lax.axis_index("c")
  s = lax.axis_index("s")
  wid = c * NS + s
  rows = pl.ds(s * RPT, RPT)
  base = wid * cpt

  @pl.when(c == 0)
  def _():
    pltpu.sync_copy(y_hbm.at[rows], acc.at[rows])

  @pl.when(c != 0)
  def _():
    pltpu.sync_copy(z_hbm.at[rows], acc.at[rows])

  pltpu.sync_copy(src_hbm.at[pl.ds(base * kc, kc)], sb0)
  pltpu.sync_copy(dst_hbm.at[pl.ds(base * kc, kc)], db0)
  plsc.subcore_barrier()
  pltpu.async_copy(y_hbm.at[sb0], msg0, gs0)
  pltpu.async_copy(src_hbm.at[pl.ds((base + 1) * kc, kc)], sb1, is1)
  pltpu.async_copy(dst_hbm.at[pl.ds((base + 1) * kc, kc)], db1, id1)

  def body(t, carry):
    j0 = 2 * t
    j1 = j0 + 1
    pltpu.make_async_copy(y_hbm.at[sb0], msg0, gs0).wait()
    pltpu.make_async_copy(src_hbm.at[pl.ds(0, kc)], sb1, is1).wait()
    pltpu.make_async_copy(dst_hbm.at[pl.ds(0, kc)], db1, id1).wait()
    pltpu.async_copy(y_hbm.at[sb1], msg1, gs1)
    pltpu.sync_copy(msg0, acc.at[db0], add=True)

    @pl.when(j0 + 2 < cpt)
    def _():
      pltpu.async_copy(src_hbm.at[pl.ds((base + j0 + 2) * kc, kc)], sb0, is0)
      pltpu.async_copy(dst_hbm.at[pl.ds((base + j0 + 2) * kc, kc)], db0, id0)

    pltpu.make_async_copy(y_hbm.at[sb1], msg1, gs1).wait()

    @pl.when(j0 + 2 < cpt)
    def _():
      pltpu.make_async_copy(src_hbm.at[pl.ds(0, kc)], sb0, is0).wait()
      pltpu.make_async_copy(dst_hbm.at[pl.ds(0, kc)], db0, id0).wait()
      pltpu.async_copy(y_hbm.at[sb0], msg0, gs0)

    pltpu.sync_copy(msg1, acc.at[db1], add=True)

    @pl.when(j1 + 2 < cpt)
    def _():
      pltpu.async_copy(src_hbm.at[pl.ds((base + j1 + 2) * kc, kc)], sb1, is1)
      pltpu.async_copy(dst_hbm.at[pl.ds((base + j1 + 2) * kc, kc)], db1, id1)

    return carry

  lax.fori_loop(0, cpt // 2, body, 0)
  plsc.subcore_barrier()
  pltpu.sync_copy(acc.at[rows], out_hbm.at[c].at[rows])


def _make_agg(d, kc):
  mesh = plsc.VectorSubcoreMesh(core_axis_name="c", subcore_axis_name="s",
                                num_cores=NC, num_subcores=NS)
  return pl.kernel(
      functools.partial(_agg_body, d, kc),
      out_type=jax.ShapeDtypeStruct((NC, NP, d), jnp.float32),
      mesh=mesh,
      scratch_types=[
          pltpu.VMEM((kc,), jnp.int32),
          pltpu.VMEM((kc,), jnp.int32),
          pltpu.VMEM((kc,), jnp.int32),
          pltpu.VMEM((kc,), jnp.int32),
          pltpu.VMEM((kc, d), jnp.float32),
          pltpu.VMEM((kc, d), jnp.float32),
          pltpu.VMEM_SHARED((NP, d), jnp.float32),
          pltpu.SemaphoreType.DMA,
          pltpu.SemaphoreType.DMA,
          pltpu.SemaphoreType.DMA,
          pltpu.SemaphoreType.DMA,
          pltpu.SemaphoreType.DMA,
          pltpu.SemaphoreType.DMA,
          pltpu.SemaphoreType.DMA,
          pltpu.SemaphoreType.DMA,
      ],
      compiler_params=pltpu.CompilerParams(use_tc_tiling_on_sc=False),
  )


def _xw_kernel(x_ref, w_ref, xw_ref):
  xw_ref[...] = jnp.dot(x_ref[...], w_ref[...],
                        preferred_element_type=jnp.float32)


def _scale_kernel(xw_ref, degp_ref, y1_ref, dis_ref):
  deg = degp_ref[0, :, 0] + degp_ref[1, :, 0] + 1.0
  dis = lax.rsqrt(deg)
  y1_ref[...] = xw_ref[...] * dis[:, None]
  dis_ref[...] = dis


def _mid_kernel(aggp_ref, dis_ref, b1_ref, w2_ref, y2_ref):
  dis = dis_ref[...]
  a = aggp_ref[0] + aggp_ref[1]
  h = jnp.maximum(a * dis[:, None] + b1_ref[...][None, :], 0.0)
  y2_ref[...] = jnp.dot(h, w2_ref[...],
                        preferred_element_type=jnp.float32) * dis[:, None]


def _out_kernel(aggp_ref, dis_ref, b2_ref, o_ref):
  a = aggp_ref[0] + aggp_ref[1]
  o_ref[...] = a * dis_ref[...][:, None] + b2_ref[...][None, :]


def _tc_xw(x, w1):
  return pl.pallas_call(
      _xw_kernel,
      grid=(GRID,),
      in_specs=[
          pl.BlockSpec((R, D_IN), lambda i: (i, 0)),
          pl.BlockSpec((D_IN, D_HID), lambda i: (0, 0)),
      ],
      out_specs=pl.BlockSpec((R, D_HID), lambda i: (i, 0)),
      out_shape=jax.ShapeDtypeStruct((NP, D_HID), jnp.float32),
  )(x, w1)


def _tc_scale(xw, degp):
  return pl.pallas_call(
      _scale_kernel,
      grid=(GRID,),
      in_specs=[
          pl.BlockSpec((R, D_HID), lambda i: (i, 0)),
          pl.BlockSpec((NC, R, 16), lambda i: (0, i, 0)),
      ],
      out_specs=[
          pl.BlockSpec((R, D_HID), lambda i: (i, 0)),
          pl.BlockSpec((R,), lambda i: (i,)),
      ],
      out_shape=[
          jax.ShapeDtypeStruct((NP, D_HID), jnp.float32),
          jax.ShapeDtypeStruct((NP,), jnp.float32),
      ],
  )(xw, degp)


def _tc_mid(aggp, dis, b1, w2):
  return pl.pallas_call(
      _mid_kernel,
      grid=(GRID,),
      in_specs=[
          pl.BlockSpec((NC, R, D_HID), lambda i: (0, i, 0)),
          pl.BlockSpec((R,), lambda i: (i,)),
          pl.BlockSpec((D_HID,), lambda i: (0,)),
          pl.BlockSpec((D_HID, D_OUT), lambda i: (0, 0)),
      ],
      out_specs=pl.BlockSpec((R, D_OUT), lambda i: (i, 0)),
      out_shape=jax.ShapeDtypeStruct((NP, D_OUT), jnp.float32),
  )(aggp, dis, b1, w2)


def _tc_out(aggp, dis, b2):
  return pl.pallas_call(
      _out_kernel,
      grid=(GRID,),
      in_specs=[
          pl.BlockSpec((NC, R, D_OUT), lambda i: (0, i, 0)),
          pl.BlockSpec((R,), lambda i: (i,)),
          pl.BlockSpec((D_OUT,), lambda i: (0,)),
      ],
      out_specs=pl.BlockSpec((R, D_OUT), lambda i: (i, 0)),
      out_shape=jax.ShapeDtypeStruct((NP, D_OUT), jnp.float32),
  )(aggp, dis, b2)


@jax.jit
def _run(x, edge_index, w1, b1, w2, b2):
  pad = (jnp.arange(EP - E, dtype=jnp.int32) % (NP - N)) + N
  src = jnp.concatenate([edge_index[0].astype(jnp.int32), pad])
  dst = jnp.concatenate([edge_index[1].astype(jnp.int32), pad])
  xp = jnp.zeros((NP, D_IN), jnp.float32).at[:N].set(x)

  ones16 = jnp.ones((512, 16), jnp.float32)
  z16 = jnp.zeros((NP, 16), jnp.float32)
  zh = jnp.zeros((NP, D_HID), jnp.float32)
  zo = jnp.zeros((NP, D_OUT), jnp.float32)

  degp = _make_deg()(dst, ones16, z16)
  xw = _tc_xw(xp, w1)
  y1, dis = _tc_scale(xw, degp)
  agg1 = _make_agg(D_HID, 512)(y1, src, dst, zh)
  y2 = _tc_mid(agg1, dis, b1, w2)
  agg2 = _make_agg(D_OUT, 128)(y2, src, dst, zo)
  out = _tc_out(agg2, dis, b2)
  return out[:N]


def kernel(x, edge_index, W1, b1, W2, b2):
  return _run(x, edge_index, W1, b1, W2, b2)


# speedup vs baseline: 31.5548x; 1.1283x over previous
import functools

import jax
import jax.numpy as jnp
from jax import lax
from jax.experimental import pallas as pl
from jax.experimental.pallas import tpu as pltpu
from jax.experimental.pallas import tpu_sc as plsc

N = 10000
E = 320000
D_IN = 128
D_HID = 64
D_OUT = 128

NC = 2
NS = 16
NW = NC * NS

KC = 512
EP = 327680
NP = 10240
RPT = NP // NS
OPT = N // NS

R = 400
GRID = N // R


def _deg_kernel(dst_hbm, ones_hbm, z_hbm, out_hbm, db0, db1, ones_v, acc,
                id0, id1):
  cpt = EP // KC // NW
  c = lax.axis_index("c")
  s = lax.axis_index("s")
  wid = c * NS + s
  base = wid * cpt
  pltpu.sync_copy(z_hbm, acc.at[pl.ds(s * RPT, RPT)])
  pltpu.sync_copy(ones_hbm, ones_v)
  pltpu.sync_copy(dst_hbm.at[pl.ds(base * KC, KC)], db0)
  plsc.subcore_barrier()
  pltpu.async_copy(dst_hbm.at[pl.ds((base + 1) * KC, KC)], db1, id1)

  def body(t, carry):
    j0 = 2 * t
    j1 = j0 + 1

    @pl.when(t > 0)
    def _():
      pltpu.make_async_copy(dst_hbm.at[pl.ds(0, KC)], db0, id0).wait()

    pltpu.sync_copy(ones_v, acc.at[db0], add=True)

    @pl.when(j0 + 2 < cpt)
    def _():
      pltpu.async_copy(dst_hbm.at[pl.ds((base + j0 + 2) * KC, KC)], db0, id0)

    pltpu.make_async_copy(dst_hbm.at[pl.ds(0, KC)], db1, id1).wait()
    pltpu.sync_copy(ones_v, acc.at[db1], add=True)

    @pl.when(j1 + 2 < cpt)
    def _():
      pltpu.async_copy(dst_hbm.at[pl.ds((base + j1 + 2) * KC, KC)], db1, id1)

    return carry

  lax.fori_loop(0, cpt // 2, body, 0)
  plsc.subcore_barrier()
  pltpu.sync_copy(acc.at[pl.ds(s * OPT, OPT)],
                  out_hbm.at[c].at[pl.ds(s * OPT, OPT)])


def _make_deg():
  mesh = plsc.VectorSubcoreMesh(core_axis_name="c", subcore_axis_name="s",
                                num_cores=NC, num_subcores=NS)
  return pl.kernel(
      _deg_kernel,
      out_type=jax.ShapeDtypeStruct((NC, N, 16), jnp.float32),
      mesh=mesh,
      scratch_types=[
          pltpu.VMEM((KC,), jnp.int32),
          pltpu.VMEM((KC,), jnp.int32),
          pltpu.VMEM((KC, 16), jnp.float32),
          pltpu.VMEM_SHARED((NP, 16), jnp.float32),
          pltpu.SemaphoreType.DMA,
          pltpu.SemaphoreType.DMA,
      ],
      compiler_params=pltpu.CompilerParams(use_tc_tiling_on_sc=False),
  )


def _acc_init(ys, z_hbm, acc, c, s, col_split):
  lo = s * RPT

  def init_from_y(ysrc):
    @pl.when(lo + RPT <= N)
    def _():
      pltpu.sync_copy(ysrc.at[pl.ds(lo, RPT)], acc.at[pl.ds(lo, RPT)])

    @pl.when(lo + RPT > N)
    def _():
      pltpu.sync_copy(ysrc.at[pl.ds(N - RPT + 240, RPT - 240)],
                      acc.at[pl.ds(lo, RPT - 240)])
      pltpu.sync_copy(z_hbm.at[pl.ds(0, 240)], acc.at[pl.ds(N, 240)])

  if col_split:
    init_from_y(ys.at[c])
  else:
    @pl.when(c == 0)
    def _():
      init_from_y(ys.at[0])

    @pl.when(c != 0)
    def _():
      pltpu.sync_copy(z_hbm, acc.at[pl.ds(lo, RPT)])


def _agg_body(col_split, ys, src_hbm, dst_hbm, z_hbm, out_hbm,
              sb0, db0, sb1, db1, msg0, msg1, acc,
              gs0, gs1, is0, id0, is1, id1):
  c = lax.axis_index("c")
  s = lax.axis_index("s")
  if col_split:
    cpt = EP // KC // NS
    base = s * cpt
    ysrc = ys.at[c]
  else:
    cpt = EP // KC // NW
    base = (c * NS + s) * cpt
    ysrc = ys.at[0]

  _acc_init(ys, z_hbm, acc, c, s, col_split)

  pltpu.sync_copy(src_hbm.at[pl.ds(base * KC, KC)], sb0)
  pltpu.sync_copy(dst_hbm.at[pl.ds(base * KC, KC)], db0)
  plsc.subcore_barrier()
  pltpu.async_copy(ysrc.at[sb0], msg0, gs0)
  pltpu.async_copy(src_hbm.at[pl.ds((base + 1) * KC, KC)], sb1, is1)
  pltpu.async_copy(dst_hbm.at[pl.ds((base + 1) * KC, KC)], db1, id1)

  def body(t, carry):
    j0 = 2 * t
    j1 = j0 + 1
    pltpu.make_async_copy(ysrc.at[sb0], msg0, gs0).wait()
    pltpu.make_async_copy(src_hbm.at[pl.ds(0, KC)], sb1, is1).wait()
    pltpu.make_async_copy(dst_hbm.at[pl.ds(0, KC)], db1, id1).wait()
    pltpu.async_copy(ysrc.at[sb1], msg1, gs1)
    pltpu.sync_copy(msg0, acc.at[db0], add=True)

    @pl.when(j0 + 2 < cpt)
    def _():
      pltpu.async_copy(src_hbm.at[pl.ds((base + j0 + 2) * KC, KC)], sb0, is0)
      pltpu.async_copy(dst_hbm.at[pl.ds((base + j0 + 2) * KC, KC)], db0, id0)

    pltpu.make_async_copy(ysrc.at[sb1], msg1, gs1).wait()

    @pl.when(j0 + 2 < cpt)
    def _():
      pltpu.make_async_copy(src_hbm.at[pl.ds(0, KC)], sb0, is0).wait()
      pltpu.make_async_copy(dst_hbm.at[pl.ds(0, KC)], db0, id0).wait()
      pltpu.async_copy(ysrc.at[sb0], msg0, gs0)

    pltpu.sync_copy(msg1, acc.at[db1], add=True)

    @pl.when(j1 + 2 < cpt)
    def _():
      pltpu.async_copy(src_hbm.at[pl.ds((base + j1 + 2) * KC, KC)], sb1, is1)
      pltpu.async_copy(dst_hbm.at[pl.ds((base + j1 + 2) * KC, KC)], db1, id1)

    return carry

  lax.fori_loop(0, cpt // 2, body, 0)
  plsc.subcore_barrier()
  pltpu.sync_copy(acc.at[pl.ds(s * OPT, OPT)],
                  out_hbm.at[c].at[pl.ds(s * OPT, OPT)])


def _make_agg(col_split):
  mesh = plsc.VectorSubcoreMesh(core_axis_name="c", subcore_axis_name="s",
                                num_cores=NC, num_subcores=NS)
  return pl.kernel(
      functools.partial(_agg_body, col_split),
      out_type=jax.ShapeDtypeStruct((NC, N, D_HID), jnp.float32),
      mesh=mesh,
      scratch_types=[
          pltpu.VMEM((KC,), jnp.int32),
          pltpu.VMEM((KC,), jnp.int32),
          pltpu.VMEM((KC,), jnp.int32),
          pltpu.VMEM((KC,), jnp.int32),
          pltpu.VMEM((KC, D_HID), jnp.float32),
          pltpu.VMEM((KC, D_HID), jnp.float32),
          pltpu.VMEM_SHARED((NP, D_HID), jnp.float32),
          pltpu.SemaphoreType.DMA,
          pltpu.SemaphoreType.DMA,
          pltpu.SemaphoreType.DMA,
          pltpu.SemaphoreType.DMA,
          pltpu.SemaphoreType.DMA,
          pltpu.SemaphoreType.DMA,
      ],
      compiler_params=pltpu.CompilerParams(use_tc_tiling_on_sc=False),
  )


def _xw_kernel(x_ref, w_ref, xw_ref):
  xw_ref[...] = jnp.dot(x_ref[...], w_ref[...],
                        preferred_element_type=jnp.float32)


def _dis(degp_ref):
  return lax.rsqrt(degp_ref[0, :, 0] + degp_ref[1, :, 0] + 1.0)


def _scale_kernel(xw_ref, degp_ref, y1_ref):
  y1_ref[0] = xw_ref[...] * _dis(degp_ref)[:, None]


def _mid_kernel(aggp_ref, degp_ref, b1_ref, w2_ref, y2_ref):
  dis = _dis(degp_ref)
  a = aggp_ref[0] + aggp_ref[1]
  h = jnp.maximum(a * dis[:, None] + b1_ref[...][None, :], 0.0)
  y2 = jnp.dot(h, w2_ref[...], preferred_element_type=jnp.float32)
  y2 = y2 * dis[:, None]
  y2_ref[0] = y2[:, :D_HID]
  y2_ref[1] = y2[:, D_HID:]


def _out_kernel(q_ref, degp_ref, b2_ref, o_ref):
  a = jnp.concatenate([q_ref[0], q_ref[1]], axis=1)
  o_ref[...] = a * _dis(degp_ref)[:, None] + b2_ref[...][None, :]


def _tc_xw(x, w1):
  return pl.pallas_call(
      _xw_kernel,
      grid=(GRID,),
      in_specs=[
          pl.BlockSpec((R, D_IN), lambda i: (i, 0)),
          pl.BlockSpec((D_IN, D_HID), lambda i: (0, 0)),
      ],
      out_specs=pl.BlockSpec((R, D_HID), lambda i: (i, 0)),
      out_shape=jax.ShapeDtypeStruct((N, D_HID), jnp.float32),
  )(x, w1)


def _tc_scale(xw, degp):
  return pl.pallas_call(
      _scale_kernel,
      grid=(GRID,),
      in_specs=[
          pl.BlockSpec((R, D_HID), lambda i: (i, 0)),
          pl.BlockSpec((NC, R, 16), lambda i: (0, i, 0)),
      ],
      out_specs=pl.BlockSpec((1, R, D_HID), lambda i: (0, i, 0)),
      out_shape=jax.ShapeDtypeStruct((1, N, D_HID), jnp.float32),
  )(xw, degp)


def _tc_mid(aggp, degp, b1, w2):
  return pl.pallas_call(
      _mid_kernel,
      grid=(GRID,),
      in_specs=[
          pl.BlockSpec((NC, R, D_HID), lambda i: (0, i, 0)),
          pl.BlockSpec((NC, R, 16), lambda i: (0, i, 0)),
          pl.BlockSpec((D_HID,), lambda i: (0,)),
          pl.BlockSpec((D_HID, D_OUT), lambda i: (0, 0)),
      ],
      out_specs=pl.BlockSpec((NC, R, D_HID), lambda i: (0, i, 0)),
      out_shape=jax.ShapeDtypeStruct((NC, N, D_HID), jnp.float32),
  )(aggp, degp, b1, w2)


def _tc_out(q, degp, b2):
  return pl.pallas_call(
      _out_kernel,
      grid=(GRID,),
      in_specs=[
          pl.BlockSpec((NC, R, D_HID), lambda i: (0, i, 0)),
          pl.BlockSpec((NC, R, 16), lambda i: (0, i, 0)),
          pl.BlockSpec((D_OUT,), lambda i: (0,)),
      ],
      out_specs=pl.BlockSpec((R, D_OUT), lambda i: (i, 0)),
      out_shape=jax.ShapeDtypeStruct((N, D_OUT), jnp.float32),
  )(q, degp, b2)


@jax.jit
def _run(x, edge_index, w1, b1, w2, b2):
  pad = jnp.arange(EP - E, dtype=jnp.int32) % 240
  src = jnp.concatenate([edge_index[0].astype(jnp.int32), pad])
  dst = jnp.concatenate([edge_index[1].astype(jnp.int32), pad + N])

  ones16 = jnp.ones((KC, 16), jnp.float32)
  z16 = jnp.zeros((RPT, 16), jnp.float32)
  zh = jnp.zeros((RPT, D_HID), jnp.float32)

  degp = _make_deg()(dst, ones16, z16)
  xw = _tc_xw(x, w1)
  y1 = _tc_scale(xw, degp)
  agg1 = _make_agg(False)(y1, src, dst, zh)
  y2 = _tc_mid(agg1, degp, b1, w2)
  agg2 = _make_agg(True)(y2, src, dst, zh)
  return _tc_out(agg2, degp, b2)


def kernel(x, edge_index, W1, b1, W2, b2):
  return _run(x, edge_index, W1, b1, W2, b2)


# speedup vs baseline: 32.4080x; 1.0270x over previous
import functools

import jax
import jax.numpy as jnp
from jax import lax
from jax.experimental import pallas as pl
from jax.experimental.pallas import tpu as pltpu
from jax.experimental.pallas import tpu_sc as plsc

N = 10000
E = 320000
D_IN = 128
D_HID = 64
D_OUT = 128

NC = 2
NS = 16
NW = NC * NS

KC = 512
EP = 327680
NP = 10240
RPT = NP // NS
OPT = N // NS

R = 400
GRID = N // R


def _deg_kernel(dst_hbm, ones_hbm, z_hbm, out_hbm, db0, db1, ones_v, acc,
                id0, id1):
  cpt = EP // KC // NW
  c = lax.axis_index("c")
  s = lax.axis_index("s")
  wid = c * NS + s
  base = wid * cpt
  pltpu.sync_copy(z_hbm, acc.at[pl.ds(s * RPT, RPT)])
  pltpu.sync_copy(ones_hbm, ones_v)
  pltpu.sync_copy(dst_hbm.at[pl.ds(base * KC, KC)], db0)
  plsc.subcore_barrier()
  pltpu.async_copy(dst_hbm.at[pl.ds((base + 1) * KC, KC)], db1, id1)

  def body(t, carry):
    j0 = 2 * t
    j1 = j0 + 1

    @pl.when(t > 0)
    def _():
      pltpu.make_async_copy(dst_hbm.at[pl.ds(0, KC)], db0, id0).wait()

    pltpu.sync_copy(ones_v, acc.at[db0], add=True)

    @pl.when(j0 + 2 < cpt)
    def _():
      pltpu.async_copy(dst_hbm.at[pl.ds((base + j0 + 2) * KC, KC)], db0, id0)

    pltpu.make_async_copy(dst_hbm.at[pl.ds(0, KC)], db1, id1).wait()
    pltpu.sync_copy(ones_v, acc.at[db1], add=True)

    @pl.when(j1 + 2 < cpt)
    def _():
      pltpu.async_copy(dst_hbm.at[pl.ds((base + j1 + 2) * KC, KC)], db1, id1)

    return carry

  lax.fori_loop(0, cpt // 2, body, 0)
  plsc.subcore_barrier()
  pltpu.sync_copy(acc.at[pl.ds(s * OPT, OPT)],
                  out_hbm.at[c].at[pl.ds(s * OPT, OPT)])


def _make_deg():
  mesh = plsc.VectorSubcoreMesh(core_axis_name="c", subcore_axis_name="s",
                                num_cores=NC, num_subcores=NS)
  return pl.kernel(
      _deg_kernel,
      out_type=jax.ShapeDtypeStruct((NC, N, 16), jnp.float32),
      mesh=mesh,
      scratch_types=[
          pltpu.VMEM((KC,), jnp.int32),
          pltpu.VMEM((KC,), jnp.int32),
          pltpu.VMEM((KC, 16), jnp.float32),
          pltpu.VMEM_SHARED((NP, 16), jnp.float32),
          pltpu.SemaphoreType.DMA,
          pltpu.SemaphoreType.DMA,
      ],
      compiler_params=pltpu.CompilerParams(use_tc_tiling_on_sc=False),
  )


def _acc_init(ys, z_hbm, acc, c, s, col_split):
  lo = s * RPT

  def init_from_y(ysrc):
    @pl.when(lo + RPT <= N)
    def _():
      pltpu.sync_copy(ysrc.at[pl.ds(lo, RPT)], acc.at[pl.ds(lo, RPT)])

    @pl.when(lo + RPT > N)
    def _():
      pltpu.sync_copy(ysrc.at[pl.ds(N - RPT + 240, RPT - 240)],
                      acc.at[pl.ds(lo, RPT - 240)])
      pltpu.sync_copy(z_hbm.at[pl.ds(0, 240)], acc.at[pl.ds(N, 240)])

  if col_split:
    init_from_y(ys.at[c])
  else:
    @pl.when(c == 0)
    def _():
      init_from_y(ys.at[0])

    @pl.when(c != 0)
    def _():
      pltpu.sync_copy(z_hbm, acc.at[pl.ds(lo, RPT)])


def _agg_body(col_split, ys, src_hbm, dst_hbm, z_hbm, out_hbm,
              sb0, db0, sb1, db1, msg0, msg1, acc,
              gs0, gs1, is0, id0, is1, id1):
  c = lax.axis_index("c")
  s = lax.axis_index("s")
  if col_split:
    cpt = EP // KC // NS
    base = s * cpt
    ysrc = ys.at[c]
  else:
    cpt = EP // KC // NW
    base = (c * NS + s) * cpt
    ysrc = ys.at[0]

  _acc_init(ys, z_hbm, acc, c, s, col_split)

  pltpu.sync_copy(src_hbm.at[pl.ds(base * KC, KC)], sb0)
  pltpu.sync_copy(dst_hbm.at[pl.ds(base * KC, KC)], db0)
  plsc.subcore_barrier()
  pltpu.async_copy(ysrc.at[sb0], msg0, gs0)
  pltpu.async_copy(src_hbm.at[pl.ds((base + 1) * KC, KC)], sb1, is1)
  pltpu.async_copy(dst_hbm.at[pl.ds((base + 1) * KC, KC)], db1, id1)

  def body(t, carry):
    j0 = 2 * t
    j1 = j0 + 1
    pltpu.make_async_copy(ysrc.at[sb0], msg0, gs0).wait()
    pltpu.make_async_copy(src_hbm.at[pl.ds(0, KC)], sb1, is1).wait()
    pltpu.make_async_copy(dst_hbm.at[pl.ds(0, KC)], db1, id1).wait()
    pltpu.async_copy(ysrc.at[sb1], msg1, gs1)
    pltpu.sync_copy(msg0, acc.at[db0], add=True)

    @pl.when(j0 + 2 < cpt)
    def _():
      pltpu.async_copy(src_hbm.at[pl.ds((base + j0 + 2) * KC, KC)], sb0, is0)
      pltpu.async_copy(dst_hbm.at[pl.ds((base + j0 + 2) * KC, KC)], db0, id0)

    pltpu.make_async_copy(ysrc.at[sb1], msg1, gs1).wait()

    @pl.when(j0 + 2 < cpt)
    def _():
      pltpu.make_async_copy(src_hbm.at[pl.ds(0, KC)], sb0, is0).wait()
      pltpu.make_async_copy(dst_hbm.at[pl.ds(0, KC)], db0, id0).wait()
      pltpu.async_copy(ysrc.at[sb0], msg0, gs0)

    pltpu.sync_copy(msg1, acc.at[db1], add=True)

    @pl.when(j1 + 2 < cpt)
    def _():
      pltpu.async_copy(src_hbm.at[pl.ds((base + j1 + 2) * KC, KC)], sb1, is1)
      pltpu.async_copy(dst_hbm.at[pl.ds((base + j1 + 2) * KC, KC)], db1, id1)

    return carry

  lax.fori_loop(0, cpt // 2, body, 0)
  plsc.subcore_barrier()
  pltpu.sync_copy(acc.at[pl.ds(s * OPT, OPT)],
                  out_hbm.at[c].at[pl.ds(s * OPT, OPT)])


def _make_agg(col_split):
  mesh = plsc.VectorSubcoreMesh(core_axis_name="c", subcore_axis_name="s",
                                num_cores=NC, num_subcores=NS)
  return pl.kernel(
      functools.partial(_agg_body, col_split),
      out_type=jax.ShapeDtypeStruct((NC, N, D_HID), jnp.float32),
      mesh=mesh,
      scratch_types=[
          pltpu.VMEM((KC,), jnp.int32),
          pltpu.VMEM((KC,), jnp.int32),
          pltpu.VMEM((KC,), jnp.int32),
          pltpu.VMEM((KC,), jnp.int32),
          pltpu.VMEM((KC, D_HID), jnp.float32),
          pltpu.VMEM((KC, D_HID), jnp.float32),
          pltpu.VMEM_SHARED((NP, D_HID), jnp.float32),
          pltpu.SemaphoreType.DMA,
          pltpu.SemaphoreType.DMA,
          pltpu.SemaphoreType.DMA,
          pltpu.SemaphoreType.DMA,
          pltpu.SemaphoreType.DMA,
          pltpu.SemaphoreType.DMA,
      ],
      compiler_params=pltpu.CompilerParams(use_tc_tiling_on_sc=False),
  )


def _mm_kernel(x_ref, w_ref, o_ref):
  o_ref[...] = jnp.dot(x_ref[...], w_ref[...],
                       preferred_element_type=jnp.float32)


def _tc_mm(x, w, din, dout):
  return pl.pallas_call(
      _mm_kernel,
      grid=(GRID,),
      in_specs=[
          pl.BlockSpec((R, din), lambda i: (i, 0)),
          pl.BlockSpec((din, dout), lambda i: (0, 0)),
      ],
      out_specs=pl.BlockSpec((R, dout), lambda i: (i, 0)),
      out_shape=jax.ShapeDtypeStruct((N, dout), jnp.float32),
  )(x, w)


@jax.jit
def _run(x, edge_index, w1, b1, w2, b2):
  pad = jnp.arange(EP - E, dtype=jnp.int32) % 240
  src = jnp.concatenate([edge_index[0].astype(jnp.int32), pad])
  dst = jnp.concatenate([edge_index[1].astype(jnp.int32), pad + N])

  ones16 = jnp.ones((KC, 16), jnp.float32)
  z16 = jnp.zeros((RPT, 16), jnp.float32)
  zh = jnp.zeros((RPT, D_HID), jnp.float32)

  degp = _make_deg()(dst, ones16, z16)
  xw = _tc_mm(x, w1, D_IN, D_HID)
  dis = lax.rsqrt(degp[0, :, 0] + degp[1, :, 0] + 1.0)
  y1 = (xw * dis[:, None])[None]
  agg1 = _make_agg(False)(y1, src, dst, zh)
  h = jnp.maximum((agg1[0] + agg1[1]) * dis[:, None] + b1[None, :], 0.0)
  y2 = _tc_mm(h, w2, D_HID, D_OUT) * dis[:, None]
  y2s = jnp.stack([y2[:, :D_HID], y2[:, D_HID:]])
  agg2 = _make_agg(True)(y2s, src, dst, zh)
  q = jnp.concatenate([agg2[0], agg2[1]], axis=1)
  return q * dis[:, None] + b2[None, :]


def kernel(x, edge_index, W1, b1, W2, b2):
  return _run(x, edge_index, W1, b1, W2, b2)


# speedup vs baseline: 33.9449x; 1.0474x over previous
import functools

import jax
import jax.numpy as jnp
from jax import lax
from jax.experimental import pallas as pl
from jax.experimental.pallas import tpu as pltpu
from jax.experimental.pallas import tpu_sc as plsc

N = 10000
E = 320000
D_IN = 128
D_HID = 64
D_OUT = 128

NC = 2
NS = 16
NW = NC * NS

KC = 512
EP = 327680
NP = 10240
RPT = NP // NS
OPT = N // NS

R = 400
GRID = N // R


def _deg_kernel(dst_hbm, ones_hbm, z_hbm, out_hbm, db0, db1, ones_v, acc,
                id0, id1):
  cpt = EP // KC // NW
  c = lax.axis_index("c")
  s = lax.axis_index("s")
  wid = c * NS + s
  base = wid * cpt
  pltpu.sync_copy(z_hbm, acc.at[pl.ds(s * RPT, RPT)])
  pltpu.sync_copy(ones_hbm, ones_v)
  pltpu.sync_copy(dst_hbm.at[pl.ds(base * KC, KC)], db0)
  plsc.subcore_barrier()
  pltpu.async_copy(dst_hbm.at[pl.ds((base + 1) * KC, KC)], db1, id1)

  def body(t, carry):
    j0 = 2 * t
    j1 = j0 + 1

    @pl.when(t > 0)
    def _():
      pltpu.make_async_copy(dst_hbm.at[pl.ds(0, KC)], db0, id0).wait()

    pltpu.sync_copy(ones_v, acc.at[db0], add=True)

    @pl.when(j0 + 2 < cpt)
    def _():
      pltpu.async_copy(dst_hbm.at[pl.ds((base + j0 + 2) * KC, KC)], db0, id0)

    pltpu.make_async_copy(dst_hbm.at[pl.ds(0, KC)], db1, id1).wait()
    pltpu.sync_copy(ones_v, acc.at[db1], add=True)

    @pl.when(j1 + 2 < cpt)
    def _():
      pltpu.async_copy(dst_hbm.at[pl.ds((base + j1 + 2) * KC, KC)], db1, id1)

    return carry

  lax.fori_loop(0, cpt // 2, body, 0)
  plsc.subcore_barrier()
  pltpu.sync_copy(acc.at[pl.ds(s * OPT, OPT)],
                  out_hbm.at[c].at[pl.ds(s * OPT, OPT)])


def _make_deg():
  mesh = plsc.VectorSubcoreMesh(core_axis_name="c", subcore_axis_name="s",
                                num_cores=NC, num_subcores=NS)
  return pl.kernel(
      _deg_kernel,
      out_type=jax.ShapeDtypeStruct((NC, N, 16), jnp.float32),
      mesh=mesh,
      scratch_types=[
          pltpu.VMEM((KC,), jnp.int32),
          pltpu.VMEM((KC,), jnp.int32),
          pltpu.VMEM((KC, 16), jnp.float32),
          pltpu.VMEM_SHARED((NP, 16), jnp.float32),
          pltpu.SemaphoreType.DMA,
          pltpu.SemaphoreType.DMA,
      ],
      compiler_params=pltpu.CompilerParams(use_tc_tiling_on_sc=False),
  )


def _acc_init(ys, z_hbm, acc, c, s, col_split):
  lo = s * RPT

  def init_from_y(ysrc):
    @pl.when(lo + RPT <= N)
    def _():
      pltpu.sync_copy(ysrc.at[pl.ds(lo, RPT)], acc.at[pl.ds(lo, RPT)])

    @pl.when(lo + RPT > N)
    def _():
      pltpu.sync_copy(ysrc.at[pl.ds(N - RPT + 240, RPT - 240)],
                      acc.at[pl.ds(lo, RPT - 240)])
      pltpu.sync_copy(z_hbm.at[pl.ds(0, 240)], acc.at[pl.ds(N, 240)])

  if col_split:
    init_from_y(ys.at[c])
  else:
    @pl.when(c == 0)
    def _():
      init_from_y(ys)

    @pl.when(c != 0)
    def _():
      pltpu.sync_copy(z_hbm, acc.at[pl.ds(lo, RPT)])


def _agg_body(col_split, ys, src_hbm, dst_hbm, z_hbm, out_hbm,
              sb0, db0, sb1, db1, msg0, msg1, acc,
              gs0, gs1, is0, id0, is1, id1):
  c = lax.axis_index("c")
  s = lax.axis_index("s")
  if col_split:
    cpt = EP // KC // NS
    base = s * cpt
    ysrc = ys.at[c]
  else:
    cpt = EP // KC // NW
    base = (c * NS + s) * cpt
    ysrc = ys

  _acc_init(ys, z_hbm, acc, c, s, col_split)

  pltpu.sync_copy(src_hbm.at[pl.ds(base * KC, KC)], sb0)
  pltpu.sync_copy(dst_hbm.at[pl.ds(base * KC, KC)], db0)
  plsc.subcore_barrier()
  pltpu.async_copy(ysrc.at[sb0], msg0, gs0)
  pltpu.async_copy(src_hbm.at[pl.ds((base + 1) * KC, KC)], sb1, is1)
  pltpu.async_copy(dst_hbm.at[pl.ds((base + 1) * KC, KC)], db1, id1)

  def body(t, carry):
    j0 = 2 * t
    j1 = j0 + 1
    pltpu.make_async_copy(ysrc.at[sb0], msg0, gs0).wait()
    pltpu.make_async_copy(src_hbm.at[pl.ds(0, KC)], sb1, is1).wait()
    pltpu.make_async_copy(dst_hbm.at[pl.ds(0, KC)], db1, id1).wait()
    pltpu.async_copy(ysrc.at[sb1], msg1, gs1)
    pltpu.sync_copy(msg0, acc.at[db0], add=True)

    @pl.when(j0 + 2 < cpt)
    def _():
      pltpu.async_copy(src_hbm.at[pl.ds((base + j0 + 2) * KC, KC)], sb0, is0)
      pltpu.async_copy(dst_hbm.at[pl.ds((base + j0 + 2) * KC, KC)], db0, id0)

    pltpu.make_async_copy(ysrc.at[sb1], msg1, gs1).wait()

    @pl.when(j0 + 2 < cpt)
    def _():
      pltpu.make_async_copy(src_hbm.at[pl.ds(0, KC)], sb0, is0).wait()
      pltpu.make_async_copy(dst_hbm.at[pl.ds(0, KC)], db0, id0).wait()
      pltpu.async_copy(ysrc.at[sb0], msg0, gs0)

    pltpu.sync_copy(msg1, acc.at[db1], add=True)

    @pl.when(j1 + 2 < cpt)
    def _():
      pltpu.async_copy(src_hbm.at[pl.ds((base + j1 + 2) * KC, KC)], sb1, is1)
      pltpu.async_copy(dst_hbm.at[pl.ds((base + j1 + 2) * KC, KC)], db1, id1)

    return carry

  lax.fori_loop(0, cpt // 2, body, 0)
  plsc.subcore_barrier()
  pltpu.sync_copy(acc.at[pl.ds(s * OPT, OPT)],
                  out_hbm.at[c].at[pl.ds(s * OPT, OPT)])


def _make_agg(col_split):
  mesh = plsc.VectorSubcoreMesh(core_axis_name="c", subcore_axis_name="s",
                                num_cores=NC, num_subcores=NS)
  return pl.kernel(
      functools.partial(_agg_body, col_split),
      out_type=jax.ShapeDtypeStruct((NC, N, D_HID), jnp.float32),
      mesh=mesh,
      scratch_types=[
          pltpu.VMEM((KC,), jnp.int32),
          pltpu.VMEM((KC,), jnp.int32),
          pltpu.VMEM((KC,), jnp.int32),
          pltpu.VMEM((KC,), jnp.int32),
          pltpu.VMEM((KC, D_HID), jnp.float32),
          pltpu.VMEM((KC, D_HID), jnp.float32),
          pltpu.VMEM_SHARED((NP, D_HID), jnp.float32),
          pltpu.SemaphoreType.DMA,
          pltpu.SemaphoreType.DMA,
          pltpu.SemaphoreType.DMA,
          pltpu.SemaphoreType.DMA,
          pltpu.SemaphoreType.DMA,
          pltpu.SemaphoreType.DMA,
      ],
      compiler_params=pltpu.CompilerParams(use_tc_tiling_on_sc=False),
  )


def _mm_kernel(x_ref, w_ref, o_ref):
  o_ref[...] = jnp.dot(x_ref[...], w_ref[...],
                       preferred_element_type=jnp.float32)


def _mid_kernel(aggp_ref, degp_ref, b1_ref, w2_ref, y2_ref):
  dis = lax.rsqrt(degp_ref[0, :, 0] + degp_ref[1, :, 0] + 1.0)
  a = aggp_ref[0] + aggp_ref[1]
  h = jnp.maximum(a * dis[:, None] + b1_ref[...][None, :], 0.0)
  y2 = jnp.dot(h, w2_ref[...], preferred_element_type=jnp.float32)
  y2 = y2 * dis[:, None]
  y2_ref[0] = y2[:, :D_HID]
  y2_ref[1] = y2[:, D_HID:]


def _tc_mid(aggp, degp, b1, w2):
  return pl.pallas_call(
      _mid_kernel,
      grid=(GRID,),
      in_specs=[
          pl.BlockSpec((NC, R, D_HID), lambda i: (0, i, 0)),
          pl.BlockSpec((NC, R, 16), lambda i: (0, i, 0)),
          pl.BlockSpec((D_HID,), lambda i: (0,)),
          pl.BlockSpec((D_HID, D_OUT), lambda i: (0, 0)),
      ],
      out_specs=pl.BlockSpec((NC, R, D_HID), lambda i: (0, i, 0)),
      out_shape=jax.ShapeDtypeStruct((NC, N, D_HID), jnp.float32),
  )(aggp, degp, b1, w2)


def _tc_mm(x, w, din, dout):
  return pl.pallas_call(
      _mm_kernel,
      grid=(GRID,),
      in_specs=[
          pl.BlockSpec((R, din), lambda i: (i, 0)),
          pl.BlockSpec((din, dout), lambda i: (0, 0)),
      ],
      out_specs=pl.BlockSpec((R, dout), lambda i: (i, 0)),
      out_shape=jax.ShapeDtypeStruct((N, dout), jnp.float32),
  )(x, w)


@jax.jit
def _run(x, edge_index, w1, b1, w2, b2):
  pad = jnp.arange(EP - E, dtype=jnp.int32) % 240
  src = jnp.concatenate([edge_index[0].astype(jnp.int32), pad])
  dst = jnp.concatenate([edge_index[1].astype(jnp.int32), pad + N])

  ones16 = jnp.ones((KC, 16), jnp.float32)
  z16 = jnp.zeros((RPT, 16), jnp.float32)
  zh = jnp.zeros((RPT, D_HID), jnp.float32)

  degp = _make_deg()(dst, ones16, z16)
  xw = _tc_mm(x, w1, D_IN, D_HID)
  dis = lax.rsqrt(degp[0, :, 0] + degp[1, :, 0] + 1.0)
  y1 = xw * dis[:, None]
  agg1 = _make_agg(False)(y1, src, dst, zh)
  y2s = _tc_mid(agg1, degp, b1, w2)
  agg2 = _make_agg(True)(y2s, src, dst, zh)
  q = jnp.concatenate([agg2[0], agg2[1]], axis=1)
  return q * dis[:, None] + b2[None, :]


def kernel(x, edge_index, W1, b1, W2, b2):
  return _run(x, edge_index, W1, b1, W2, b2)
